# baseline (speedup 1.0000x reference)
import functools

import jax
import jax.numpy as jnp
from jax import lax
from jax.experimental import pallas as pl
from jax.experimental.pallas import tpu as pltpu
from jax.experimental.pallas import tpu_sc as plsc

_LANES = 16
_K = 8


def _newton_rsqrt(a):
    i = plsc.bitcast(a, jnp.int32)
    y = plsc.bitcast(jnp.int32(0x5F3759DF) - (i >> 1), jnp.float32)
    for _ in range(3):
        y = y * (1.5 - (0.5 * a) * y * y)
    return y


def _make_sc_gather(B, T_pad):
    mesh = plsc.VectorSubcoreMesh(core_axis_name="c", subcore_axis_name="s")
    n_chunks = B // _LANES
    f32 = jnp.float32

    @functools.partial(
        pl.kernel,
        out_type=(
            jax.ShapeDtypeStruct((B,), f32),
            jax.ShapeDtypeStruct((B,), f32),
        ),
        mesh=mesh,
        scratch_types=[
            pltpu.VMEM((T_pad,), f32),
            pltpu.VMEM((_LANES,), jnp.int32),
            pltpu.VMEM((_LANES,), f32),
            pltpu.VMEM((_LANES,), f32),
        ],
        compiler_params=pltpu.CompilerParams(needs_layout_passes=False),
    )
    def sc_gather(t_hbm, ab_hbm, c1_hbm, c2_hbm, table_v, t_v, c1_v, c2_v):
        w = lax.axis_index("s") * 2 + lax.axis_index("c")

        @pl.when(w < n_chunks)
        def _():
            base = w * _LANES
            pltpu.sync_copy(ab_hbm, table_v)
            pltpu.sync_copy(t_hbm.at[pl.ds(base, _LANES)], t_v)
            a = plsc.load_gather(table_v, [t_v[...]])
            c1_v[...] = _newton_rsqrt(a)
            c2_v[...] = _newton_rsqrt(a / (1.0 - a))
            pltpu.sync_copy(c1_v, c1_hbm.at[pl.ds(base, _LANES)])
            pltpu.sync_copy(c2_v, c2_hbm.at[pl.ds(base, _LANES)])

    return sc_gather


def _tc_body(c1_ref, c2_ref, x_hbm, n_hbm, o_hbm, xb, nb, ob, sx, sn, so):
    nch = pl.num_programs(0)
    i = pl.program_id(0)
    slot = lax.rem(i, _K)

    @pl.when(i == 0)
    def _prologue():
        for j in range(_K):
            pltpu.make_async_copy(x_hbm.at[j], xb.at[j], sx.at[j]).start()
            pltpu.make_async_copy(n_hbm.at[j], nb.at[j], sn.at[j]).start()

    pltpu.make_async_copy(x_hbm.at[i], xb.at[slot], sx.at[slot]).wait()
    pltpu.make_async_copy(n_hbm.at[i], nb.at[slot], sn.at[slot]).wait()

    @pl.when(i >= _K)
    def _drain_out():
        pltpu.make_async_copy(ob.at[slot], o_hbm.at[i - _K], so.at[slot]).wait()

    ob.at[slot][...] = c1_ref[i] * xb.at[slot][...] - c2_ref[i] * nb.at[slot][...]
    pltpu.make_async_copy(ob.at[slot], o_hbm.at[i], so.at[slot]).start()

    @pl.when(i + _K < nch)
    def _prefetch():
        pltpu.make_async_copy(x_hbm.at[i + _K], xb.at[slot], sx.at[slot]).start()
        pltpu.make_async_copy(n_hbm.at[i + _K], nb.at[slot], sn.at[slot]).start()

    @pl.when(i == nch - 1)
    def _epilogue():
        for j in range(_K):
            pltpu.make_async_copy(
                ob.at[j], o_hbm.at[nch - _K + j], so.at[j]
            ).wait()


def kernel(x_t, t, pred_noise, alphas_bar):
    B, C, H, W = x_t.shape
    T = alphas_bar.shape[0]

    T_pad = (T + 255) // 256 * 256
    ab = jnp.concatenate([alphas_bar, jnp.ones((T_pad - T,), jnp.float32)])
    c1, c2 = _make_sc_gather(B, T_pad)(t, ab)

    out = pl.pallas_call(
        _tc_body,
        grid=(B,),
        in_specs=[
            pl.BlockSpec(memory_space=pltpu.SMEM),
            pl.BlockSpec(memory_space=pltpu.SMEM),
            pl.BlockSpec(memory_space=pltpu.MemorySpace.HBM),
            pl.BlockSpec(memory_space=pltpu.MemorySpace.HBM),
        ],
        out_specs=pl.BlockSpec(memory_space=pltpu.MemorySpace.HBM),
        out_shape=jax.ShapeDtypeStruct((B, C, H, W), jnp.float32),
        scratch_shapes=[
            pltpu.VMEM((_K, C, H, W), jnp.float32),
            pltpu.VMEM((_K, C, H, W), jnp.float32),
            pltpu.VMEM((_K, C, H, W), jnp.float32),
            pltpu.SemaphoreType.DMA((_K,)),
            pltpu.SemaphoreType.DMA((_K,)),
            pltpu.SemaphoreType.DMA((_K,)),
        ],
    )(c1, c2, x_t, pred_noise)

    return out


# speedup vs baseline: 1.4155x; 1.4155x over previous
import jax
import jax.numpy as jnp
from jax import lax
from jax.experimental import pallas as pl
from jax.experimental.pallas import tpu as pltpu

_K = 16


def _coeffs(t_ref, ab_ref, i):
    a = ab_ref[t_ref[i]]
    return jax.lax.rsqrt(a), jnp.sqrt(1.0 / a - 1.0)


def _body(t_ref, ab_ref, x_hbm, n_hbm, o_hbm, xb, nb, ob, sx, sn, so):
    nch = pl.num_programs(0)
    i = pl.program_id(0)
    slot = lax.rem(i, _K)

    @pl.when(i == 0)
    def _prologue():
        for j in range(_K):
            pltpu.make_async_copy(x_hbm.at[j], xb.at[j], sx.at[j]).start()
            pltpu.make_async_copy(n_hbm.at[j], nb.at[j], sn.at[j]).start()

    pltpu.make_async_copy(x_hbm.at[i], xb.at[slot], sx.at[slot]).wait()
    pltpu.make_async_copy(n_hbm.at[i], nb.at[slot], sn.at[slot]).wait()

    @pl.when(i >= _K)
    def _drain_out():
        pltpu.make_async_copy(ob.at[slot], o_hbm.at[i - _K], so.at[slot]).wait()

    c1, c2 = _coeffs(t_ref, ab_ref, i)
    ob.at[slot][...] = c1 * xb.at[slot][...] - c2 * nb.at[slot][...]
    pltpu.make_async_copy(ob.at[slot], o_hbm.at[i], so.at[slot]).start()

    @pl.when(i + _K < nch)
    def _prefetch():
        pltpu.make_async_copy(x_hbm.at[i + _K], xb.at[slot], sx.at[slot]).start()
        pltpu.make_async_copy(n_hbm.at[i + _K], nb.at[slot], sn.at[slot]).start()

    @pl.when(i == nch - 1)
    def _epilogue():
        for j in range(_K):
            pltpu.make_async_copy(
                ob.at[j], o_hbm.at[nch - _K + j], so.at[j]
            ).wait()


def kernel(x_t, t, pred_noise, alphas_bar):
    B, C, H, W = x_t.shape

    out = pl.pallas_call(
        _body,
        grid=(B,),
        in_specs=[
            pl.BlockSpec(memory_space=pltpu.SMEM),
            pl.BlockSpec(memory_space=pltpu.SMEM),
            pl.BlockSpec(memory_space=pltpu.MemorySpace.HBM),
            pl.BlockSpec(memory_space=pltpu.MemorySpace.HBM),
        ],
        out_specs=pl.BlockSpec(memory_space=pltpu.MemorySpace.HBM),
        out_shape=jax.ShapeDtypeStruct((B, C, H, W), jnp.float32),
        scratch_shapes=[
            pltpu.VMEM((_K, C, H, W), jnp.float32),
            pltpu.VMEM((_K, C, H, W), jnp.float32),
            pltpu.VMEM((_K, C, H, W), jnp.float32),
            pltpu.SemaphoreType.DMA((_K,)),
            pltpu.SemaphoreType.DMA((_K,)),
            pltpu.SemaphoreType.DMA((_K,)),
        ],
    )(t, alphas_bar, x_t, pred_noise)

    return out


# speedup vs baseline: 1.4255x; 1.0071x over previous
import jax
import jax.numpy as jnp
from jax import lax
from jax.experimental import pallas as pl
from jax.experimental.pallas import tpu as pltpu

_K = 8


def _coeffs(t_ref, ab_ref, i):
    a = ab_ref[t_ref[i]]
    return jax.lax.rsqrt(a), jnp.sqrt(1.0 / a - 1.0)


def _body(t_ref, ab_ref, x_hbm, n_hbm, o_hbm, xb, nb, ob, sx, sn, so):
    nch = pl.num_programs(0)
    i = pl.program_id(0)
    slot = lax.rem(i, _K)

    @pl.when(i == 0)
    def _prologue():
        for j in range(_K):
            pltpu.make_async_copy(x_hbm.at[j], xb.at[j], sx.at[j]).start()
            pltpu.make_async_copy(n_hbm.at[j], nb.at[j], sn.at[j]).start()

    pltpu.make_async_copy(x_hbm.at[i], xb.at[slot], sx.at[slot]).wait()
    pltpu.make_async_copy(n_hbm.at[i], nb.at[slot], sn.at[slot]).wait()

    @pl.when(i >= _K)
    def _drain_out():
        pltpu.make_async_copy(ob.at[slot], o_hbm.at[i - _K], so.at[slot]).wait()

    c1, c2 = _coeffs(t_ref, ab_ref, i)
    ob.at[slot][...] = c1 * xb.at[slot][...] - c2 * nb.at[slot][...]
    pltpu.make_async_copy(ob.at[slot], o_hbm.at[i], so.at[slot]).start()

    @pl.when(i + _K < nch)
    def _prefetch():
        pltpu.make_async_copy(x_hbm.at[i + _K], xb.at[slot], sx.at[slot]).start()
        pltpu.make_async_copy(n_hbm.at[i + _K], nb.at[slot], sn.at[slot]).start()

    @pl.when(i == nch - 1)
    def _epilogue():
        for j in range(_K):
            pltpu.make_async_copy(
                ob.at[j], o_hbm.at[nch - _K + j], so.at[j]
            ).wait()


def kernel(x_t, t, pred_noise, alphas_bar):
    B, C, H, W = x_t.shape

    out = pl.pallas_call(
        _body,
        grid=(B,),
        in_specs=[
            pl.BlockSpec(memory_space=pltpu.SMEM),
            pl.BlockSpec(memory_space=pltpu.SMEM),
            pl.BlockSpec(memory_space=pltpu.MemorySpace.HBM),
            pl.BlockSpec(memory_space=pltpu.MemorySpace.HBM),
        ],
        out_specs=pl.BlockSpec(memory_space=pltpu.MemorySpace.HBM),
        out_shape=jax.ShapeDtypeStruct((B, C, H, W), jnp.float32),
        scratch_shapes=[
            pltpu.VMEM((_K, C, H, W), jnp.float32),
            pltpu.VMEM((_K, C, H, W), jnp.float32),
            pltpu.VMEM((_K, C, H, W), jnp.float32),
            pltpu.SemaphoreType.DMA((_K,)),
            pltpu.SemaphoreType.DMA((_K,)),
            pltpu.SemaphoreType.DMA((_K,)),
        ],
    )(t, alphas_bar, x_t, pred_noise)

    return out
